# baseline (speedup 1.0000x reference)
import functools

import jax
import jax.numpy as jnp
from jax import lax
from jax.experimental import pallas as pl
from jax.experimental.pallas import tpu as pltpu
from jax.experimental.pallas import tpu_sc as plsc

_N_SLICES = 1024
_NUM_FREQS = 64
_NY = 24
_NX = 24
_COORD_DIM = 4
_B = 16384
_NYQUIST = 1000.0
_P = _NY * _NX
_NC, _NS = 2, 16
_NW = _NC * _NS
_PPW = _P // _NW
_BPW = _B // _NW
_NG = _B // 16


def _body(tab_hbm, coords_hbm, freq_hbm, idx_hbm, out_hbm, lab_hbm,
          idx_v, slab_v, row_v, coords_v, freq_v, lab_v,
          slab_sem, row_sem0, row_sem1, lab_sem):
    wid = lax.axis_index("s") * _NC + lax.axis_index("c")
    row_sems = (row_sem0, row_sem1)

    def start_slab(p):
        y = p // _NX
        xt = (p % _NX) // 8
        xi = p % 8
        return pltpu.async_copy(
            tab_hbm.at[:, y, xt, :, xi, :], slab_v, slab_sem)

    p0 = wid * _PPW
    slab_cp = start_slab(p0)

    stage_cps = [
        pltpu.async_copy(idx_hbm, idx_v, row_sem0),
        pltpu.async_copy(coords_hbm, coords_v, row_sem1),
        pltpu.async_copy(freq_hbm, freq_v, lab_sem),
    ]
    for cp in stage_cps:
        cp.wait()

    base = wid * _BPW
    for q in range(_BPW // 16):
        raw = idx_v[pl.ds(base + q * 16, 16)]
        f = raw & (_NUM_FREQS - 1)
        sb = raw >> 13
        sl = (raw >> 6) & 127
        blk, col = q // 8, (q % 8) * 16
        for c in range(_COORD_DIM):
            lab_v[blk, c, pl.ds(col, 16)] = plsc.load_gather(
                coords_v, [sb, jnp.full((16,), c, jnp.int32), sl])
        lab_v[blk, _COORD_DIM, pl.ds(col, 16)] = (
            plsc.load_gather(freq_v, [f]) * (1.0 / _NYQUIST))
    lab_cp = pltpu.async_copy(
        lab_v, lab_hbm.at[pl.ds(wid * (_BPW // 128), _BPW // 128)], lab_sem)

    def gather_row(row_ref):
        @plsc.parallel_loop(0, _NG, 1, unroll=16)
        def _(i):
            raw = idx_v[pl.ds(i * 16, 16)]
            vals = plsc.load_gather(
                slab_v,
                [raw & (_NUM_FREQS - 1), raw >> 13, (raw >> 6) & 127])
            row_ref[i >> 3, pl.ds((i & 7) * 16, 16)] = vals

    row_cp = [None] * _PPW
    for j in range(_PPW):
        p = p0 + j
        slab_cp.wait()
        if j >= 2:
            row_cp[j - 2].wait()
        gather_row(row_v.at[j % 2])
        if j + 1 < _PPW:
            slab_cp = start_slab(p + 1)
        row_cp[j] = pltpu.async_copy(
            row_v.at[j % 2], out_hbm.at[p // 8, :, p % 8, :],
            row_sems[j % 2])
    row_cp[_PPW - 2].wait()
    row_cp[_PPW - 1].wait()
    lab_cp.wait()


_sc_call = functools.partial(
    pl.kernel,
    out_type=(
        jax.ShapeDtypeStruct((_P // 8, _B // 128, 8, 128), jnp.float32),
        jax.ShapeDtypeStruct((_B // 128, 8, 128), jnp.float32),
    ),
    mesh=plsc.VectorSubcoreMesh(core_axis_name="c", subcore_axis_name="s"),
    scratch_types=[
        pltpu.VMEM((_B,), jnp.int32),
        pltpu.VMEM((_NUM_FREQS, 8, 128), jnp.float32),
        pltpu.VMEM((2, _B // 128, 128), jnp.float32),
        pltpu.VMEM((8, _COORD_DIM, 128), jnp.float32),
        pltpu.VMEM((_NUM_FREQS,), jnp.float32),
        pltpu.VMEM((_BPW // 128, 8, 128), jnp.float32),
        pltpu.SemaphoreType.DMA,
        pltpu.SemaphoreType.DMA,
        pltpu.SemaphoreType.DMA,
        pltpu.SemaphoreType.DMA,
    ],
    compiler_params=pltpu.CompilerParams(
        needs_layout_passes=False, use_tc_tiling_on_sc=False),
)(_body)


def kernel(slices, coords, freq_algn, indices):
    tab6 = (slices.transpose(1, 2, 3, 0)
            .reshape(_NUM_FREQS, _NY, _NX // 8, 8, _N_SLICES // 128, 128)
            .transpose(0, 1, 2, 4, 3, 5))
    coords3 = coords.transpose(1, 0).reshape(
        _COORD_DIM, _N_SLICES // 128, 128).transpose(1, 0, 2)
    out6, lab6 = _sc_call(tab6, coords3, freq_algn,
                          indices.astype(jnp.int32))
    samples = (out6.reshape(_NY, _NX // 8, _B // 128, 8, 128)
               .transpose(2, 4, 0, 1, 3)
               .reshape(_B, _NY, _NX)[:, None, :, :])
    labels = lab6.transpose(0, 2, 1).reshape(_B, 8)[:, :_COORD_DIM + 1]
    return samples, labels
